# Initial kernel scaffold; baseline (speedup 1.0000x reference)
import jax
import jax.numpy as jnp
from jax.experimental import pallas as pl


def kernel(metadata, deltas, proposals, scores, masks):
    raise NotImplementedError("write your pallas kernel here")


# speedup vs baseline: 57.4761x; 57.4761x over previous
import functools

import jax
import jax.numpy as jnp
from jax import lax
from jax.experimental import pallas as pl
from jax.experimental.pallas import tpu as pltpu

_N = 20000
_C = 21
_PAD = 300
_THR = 0.5
_KMAX = 512
_MD = 14 * 14 * 21


def _stage_a(meta_ref, prop_ref, dx_ref, dy_ref, dw_ref, dh_ref, sc_ref,
             boxes_ref, msc_ref):
    img_h = meta_ref[0, 0]
    img_w = meta_ref[0, 1]
    scale = meta_ref[0, 2]
    prop = prop_ref[...] / scale
    x1 = prop[:, 0]
    y1 = prop[:, 1]
    w = prop[:, 2] - x1 + 1.0
    h = prop[:, 3] - y1 + 1.0
    cx = x1 + 0.5 * w
    cy = y1 + 0.5 * h

    sc = sc_ref[...]
    nb = sc.shape[0]
    top = jnp.argmax(sc, axis=1)
    oh = lax.broadcasted_iota(jnp.int32, (nb, _C), 1) == top[:, None]
    zf = jnp.zeros((nb, _C), jnp.float32)
    dx = jnp.sum(jnp.where(oh, dx_ref[...], zf), axis=1)
    dy = jnp.sum(jnp.where(oh, dy_ref[...], zf), axis=1)
    dw = jnp.sum(jnp.where(oh, dw_ref[...], zf), axis=1)
    dh = jnp.sum(jnp.where(oh, dh_ref[...], zf), axis=1)

    pcx = dx * w + cx
    pcy = dy * h + cy
    pw = jnp.exp(dw) * w
    ph = jnp.exp(dh) * h
    px1 = jnp.clip(pcx - 0.5 * pw, 0.0, img_w - 1.0)
    py1 = jnp.clip(pcy - 0.5 * ph, 0.0, img_h - 1.0)
    px2 = jnp.clip(pcx + 0.5 * pw, 0.0, img_w - 1.0)
    py2 = jnp.clip(pcy + 0.5 * ph, 0.0, img_h - 1.0)

    boxes_ref[...] = jnp.stack([px1, py1, px2, py2], axis=1)
    msc_ref[...] = jnp.max(sc[:, 1:], axis=1)[:, None]


def _nms(bs_ref, sel_ref, cnt_ref):
    lanes = lax.broadcasted_iota(jnp.int32, (1, _KMAX), 1)
    big = jnp.float32(3e8)
    kx1 = jnp.full((1, _KMAX), big, jnp.float32)
    ky1 = jnp.full((1, _KMAX), big, jnp.float32)
    kx2 = jnp.full((1, _KMAX), -big, jnp.float32)
    ky2 = jnp.full((1, _KMAX), -big, jnp.float32)
    kar = (kx2 - kx1 + 1.0) * (ky2 - ky1 + 1.0)
    sel = jnp.zeros((1, _KMAX), jnp.int32)

    def cond(st):
        i, cnt = st[0], st[1]
        return (i < _N) & (cnt < _PAD)

    def body(st):
        i, cnt, kx1, ky1, kx2, ky2, kar, sel = st
        row = bs_ref[pl.ds(i, 1), :]
        bx1 = jnp.broadcast_to(row[:, 0:1], (1, _KMAX))
        by1 = jnp.broadcast_to(row[:, 1:2], (1, _KMAX))
        bx2 = jnp.broadcast_to(row[:, 2:3], (1, _KMAX))
        by2 = jnp.broadcast_to(row[:, 3:4], (1, _KMAX))
        ar_i = (bx2 - bx1 + 1.0) * (by2 - by1 + 1.0)
        xx1 = jnp.maximum(bx1, kx1)
        yy1 = jnp.maximum(by1, ky1)
        xx2 = jnp.minimum(bx2, kx2)
        yy2 = jnp.minimum(by2, ky2)
        iw = jnp.maximum(xx2 - xx1 + 1.0, 0.0)
        ih = jnp.maximum(yy2 - yy1 + 1.0, 0.0)
        inter = iw * ih
        iou = inter / (ar_i + kar - inter)
        keepit = ~jnp.any(iou > _THR)
        ins = (lanes == cnt) & keepit
        kx1 = jnp.where(ins, bx1, kx1)
        ky1 = jnp.where(ins, by1, ky1)
        kx2 = jnp.where(ins, bx2, kx2)
        ky2 = jnp.where(ins, by2, ky2)
        kar = jnp.where(ins, ar_i, kar)
        sel = jnp.where(ins, i, sel)
        return (i + 1, cnt + keepit.astype(jnp.int32), kx1, ky1, kx2, ky2,
                kar, sel)

    st = lax.while_loop(
        cond, body,
        (jnp.int32(0), jnp.int32(0), kx1, ky1, kx2, ky2, kar, sel))
    sel_ref[...] = st[7]
    cnt_ref[0, 0] = st[1]


def _gather(idx_ref, cnt_ref, boxes_ref, sc_ref, m_ref, ob_ref, os_ref,
            om_ref):
    j = pl.program_id(0)
    vf = (j < cnt_ref[0]).astype(jnp.float32)
    ob_ref[...] = boxes_ref[...] * vf
    os_ref[...] = sc_ref[...] * vf
    om_ref[...] = m_ref[...] * vf


def kernel(metadata, deltas, proposals, scores, masks):
    prop = proposals[0]
    d4 = deltas[0].reshape(_N, _C, 4)
    sc = scores[0]

    bn = 2000
    boxes_top, msc = pl.pallas_call(
        _stage_a,
        grid=(_N // bn,),
        in_specs=[
            pl.BlockSpec(memory_space=pltpu.SMEM),
            pl.BlockSpec((bn, 4), lambda i: (i, 0)),
            pl.BlockSpec((bn, _C), lambda i: (i, 0)),
            pl.BlockSpec((bn, _C), lambda i: (i, 0)),
            pl.BlockSpec((bn, _C), lambda i: (i, 0)),
            pl.BlockSpec((bn, _C), lambda i: (i, 0)),
            pl.BlockSpec((bn, _C), lambda i: (i, 0)),
        ],
        out_specs=[
            pl.BlockSpec((bn, 4), lambda i: (i, 0)),
            pl.BlockSpec((bn, 1), lambda i: (i, 0)),
        ],
        out_shape=[
            jax.ShapeDtypeStruct((_N, 4), jnp.float32),
            jax.ShapeDtypeStruct((_N, 1), jnp.float32),
        ],
    )(metadata, prop, d4[..., 0], d4[..., 1], d4[..., 2], d4[..., 3], sc)

    order = jnp.argsort(-msc[:, 0])
    bs = boxes_top[order]

    sel, cnt = pl.pallas_call(
        _nms,
        in_specs=[pl.BlockSpec(memory_space=pltpu.VMEM)],
        out_specs=[
            pl.BlockSpec(memory_space=pltpu.VMEM),
            pl.BlockSpec(memory_space=pltpu.SMEM),
        ],
        out_shape=[
            jax.ShapeDtypeStruct((1, _KMAX), jnp.int32),
            jax.ShapeDtypeStruct((1, 1), jnp.int32),
        ],
    )(bs)

    idx = order[sel[0, :_PAD]]
    cnt1 = cnt.reshape(1)

    grid_spec = pltpu.PrefetchScalarGridSpec(
        num_scalar_prefetch=2,
        grid=(_PAD,),
        in_specs=[
            pl.BlockSpec((1, 1, 4), lambda j, idx, cnt: (idx[j], 0, 0)),
            pl.BlockSpec((1, 1, _C), lambda j, idx, cnt: (idx[j], 0, 0)),
            pl.BlockSpec((1, 1, _MD), lambda j, idx, cnt: (idx[j], 0, 0)),
        ],
        out_specs=[
            pl.BlockSpec((1, 1, 4), lambda j, idx, cnt: (j, 0, 0)),
            pl.BlockSpec((1, 1, _C), lambda j, idx, cnt: (j, 0, 0)),
            pl.BlockSpec((1, 1, _MD), lambda j, idx, cnt: (j, 0, 0)),
        ],
    )
    ob, osc, om = pl.pallas_call(
        _gather,
        grid_spec=grid_spec,
        out_shape=[
            jax.ShapeDtypeStruct((_PAD, 1, 4), jnp.float32),
            jax.ShapeDtypeStruct((_PAD, 1, _C), jnp.float32),
            jax.ShapeDtypeStruct((_PAD, 1, _MD), jnp.float32),
        ],
    )(idx, cnt1, boxes_top.reshape(_N, 1, 4), sc.reshape(_N, 1, _C),
      masks[0].reshape(_N, 1, _MD))

    return (ob.reshape(1, _PAD, 4), osc.reshape(1, _PAD, _C),
            om.reshape(1, _PAD, 14, 14, 21))
